# baseline (speedup 1.0000x reference)
import dataclasses
import functools

import jax
import jax.numpy as jnp
from jax.experimental import pallas as pl
from jax.experimental.pallas import tpu as pltpu
from jax.experimental.pallas import tpu_sc as plsc

_D = 156
_PERIOD = 624
_NVEC = _PERIOD // 16
_BLK = 19968
_GROUPS = _BLK // _PERIOD


def _sc_body(idx_hbm, x_hbm, o_hbm, idx_vmem):
    pltpu.sync_copy(idx_hbm, idx_vmem)
    total = x_hbm.shape[0]

    def block_body(x_vmem, o_vmem):
        iv = [idx_vmem[pl.ds(16 * v, 16)] for v in range(_NVEC)]

        @pl.loop(0, _GROUPS)
        def _(g):
            base = g * _PERIOD
            for v in range(_NVEC):
                off = base + 16 * v
                gv = plsc.load_gather(x_vmem, [iv[v] + base])
                o_vmem[pl.ds(off, 16)] = x_vmem[pl.ds(off, 16)] - gv

    pltpu.emit_pipeline(
        block_body,
        grid=(total // _BLK,),
        in_specs=[pl.BlockSpec((_BLK,), lambda i: (i,))],
        out_specs=[pl.BlockSpec((_BLK,), lambda i: (i,))],
        core_axis_name=("c", "s"),
        dimension_semantics=(pltpu.PARALLEL,),
    )(x_hbm, o_hbm)


@jax.jit
def kernel(positions, parents):
    B, T, J, C = positions.shape
    total = B * T * J * C
    xf = positions.reshape(total)

    perm = (parents.astype(jnp.int32)[:, None] * C
            + jnp.arange(C, dtype=jnp.int32)[None, :]).reshape(_D)
    idx = (jnp.arange(_PERIOD // _D, dtype=jnp.int32)[:, None] * _D
           + perm[None, :]).reshape(_PERIOD)

    cp = pltpu.CompilerParams()
    if "needs_layout_passes" in pltpu.CompilerParams.__dataclass_fields__:
        cp = dataclasses.replace(cp, needs_layout_passes=False)
    sc_call = pl.kernel(
        _sc_body,
        out_type=jax.ShapeDtypeStruct((total,), jnp.float32),
        mesh=plsc.VectorSubcoreMesh(core_axis_name="c", subcore_axis_name="s"),
        scratch_types=[pltpu.VMEM((_PERIOD,), jnp.int32)],
        compiler_params=cp,
    )
    return sc_call(idx, xf).reshape(B, T, J, C)


# speedup vs baseline: 498.1347x; 498.1347x over previous
import jax
import jax.numpy as jnp
from jax import lax
from jax.experimental import pallas as pl
from jax.experimental.pallas import tpu as pltpu


def _offsets_body(perm_ref, x_ref, o_ref):
    def step(i, carry):
        p = perm_ref[i]
        o_ref[i] = x_ref[i] - x_ref[p]
        return carry

    lax.fori_loop(0, x_ref.shape[0], step, 0)


@jax.jit
def kernel(positions, parents):
    B, T, J, C = positions.shape
    D = J * C
    x = positions.transpose(2, 3, 0, 1).reshape(D, B, T)

    perm = (parents.astype(jnp.int32)[:, None] * C
            + jnp.arange(C, dtype=jnp.int32)[None, :]).reshape(D)

    RB, CT = 8, 1024
    out = pl.pallas_call(
        _offsets_body,
        grid_spec=pltpu.PrefetchScalarGridSpec(
            num_scalar_prefetch=1,
            grid=(B // RB, T // CT),
            in_specs=[pl.BlockSpec((D, RB, CT), lambda i, j, perm_ref: (0, i, j))],
            out_specs=pl.BlockSpec((D, RB, CT), lambda i, j, perm_ref: (0, i, j)),
        ),
        out_shape=jax.ShapeDtypeStruct((D, B, T), jnp.float32),
    )(perm, x)
    return out.reshape(J, C, B, T).transpose(2, 3, 0, 1)


# speedup vs baseline: 517.4539x; 1.0388x over previous
import jax
import jax.numpy as jnp
from jax import lax
from jax.experimental import pallas as pl
from jax.experimental.pallas import tpu as pltpu


def _offsets_body(perm_ref, x_ref, o_ref):
    def step(i, carry):
        p = perm_ref[i]
        o_ref[i] = x_ref[i] - x_ref[p]
        return carry

    lax.fori_loop(0, x_ref.shape[0], step, 0)


@jax.jit
def kernel(positions, parents):
    B, T, J, C = positions.shape
    D = J * C
    x = positions.transpose(2, 3, 0, 1).reshape(D, B, T)

    perm = (parents.astype(jnp.int32)[:, None] * C
            + jnp.arange(C, dtype=jnp.int32)[None, :]).reshape(D)

    RB, CT = 8, 2048
    out = pl.pallas_call(
        _offsets_body,
        grid_spec=pltpu.PrefetchScalarGridSpec(
            num_scalar_prefetch=1,
            grid=(B // RB, T // CT),
            in_specs=[pl.BlockSpec((D, RB, CT), lambda i, j, perm_ref: (0, i, j))],
            out_specs=pl.BlockSpec((D, RB, CT), lambda i, j, perm_ref: (0, i, j)),
        ),
        out_shape=jax.ShapeDtypeStruct((D, B, T), jnp.float32),
    )(perm, x)
    return out.reshape(J, C, B, T).transpose(2, 3, 0, 1)
